# baseline (speedup 1.0000x reference)
import functools

import jax
import jax.numpy as jnp
from jax import lax
from jax.experimental import pallas as pl
from jax.experimental.pallas import tpu as pltpu
from jax.experimental.pallas import tpu_sc as plsc

NUM_CLASSES = 100000
NUM_INTERNAL = NUM_CLASSES - 1
DEPTH = 17
FEAT = 128
LANES = 16
CHUNK = 16
NGATH = 9
NHOT = 256
NFC = FEAT // LANES


def _sc_partials(features, tgt, wdiff, batch):
    info = plsc.get_sparse_core_info()
    nc, ns = info.num_cores, info.num_subcores
    nw = nc * ns
    spw = batch // nw
    nchunk = spw // CHUNK
    na = 8 * CHUNK
    nb = (NGATH - 8) * CHUNK

    mesh = plsc.VectorSubcoreMesh(core_axis_name="c", subcore_axis_name="s")

    @functools.partial(
        pl.kernel,
        mesh=mesh,
        out_type=jax.ShapeDtypeStruct((batch, DEPTH * LANES), jnp.float32),
        scratch_types=[
            pltpu.VMEM((spw,), jnp.int32),
            pltpu.VMEM((2, na), jnp.int32),
            pltpu.VMEM((2, nb), jnp.int32),
            pltpu.VMEM((na, FEAT), jnp.float32),
            pltpu.VMEM((nb, FEAT), jnp.float32),
            pltpu.VMEM((NHOT, FEAT), jnp.float32),
            pltpu.VMEM((CHUNK, FEAT), jnp.float32),
            pltpu.VMEM((CHUNK, DEPTH * LANES), jnp.float32),
            pltpu.SemaphoreType.DMA,
            pltpu.SemaphoreType.DMA,
        ],
    )
    def sc_fn(feat_hbm, tgt_hbm, table_hbm, out_hbm,
              tgt_v, idx_a, idx_b, rows_a, rows_b, hot_v, fd_v, out_v,
              sem_a, sem_b):
        wid = lax.axis_index("s") * nc + lax.axis_index("c")
        base = wid * spw
        pltpu.sync_copy(tgt_hbm.at[pl.ds(base, spw)], tgt_v)
        pltpu.sync_copy(table_hbm.at[pl.ds(0, NHOT)], hot_v)

        def store_idx(c, p):
            q = tgt_v[pl.ds(c * CHUNK, CHUNK)] + NUM_CLASSES
            for d in range(NGATH):
                node = jnp.maximum((q >> (d + 1)) - 1, 0)
                if d < 8:
                    idx_a[p, pl.ds(d * CHUNK, CHUNK)] = node
                else:
                    idx_b[p, pl.ds((d - 8) * CHUNK, CHUNK)] = node

        def fire_a(p):
            return pltpu.async_copy(table_hbm.at[idx_a.at[p]], rows_a, sem_a)

        def fire_b(p):
            return pltpu.async_copy(table_hbm.at[idx_b.at[p]], rows_b, sem_b)

        def tree_sum(parts):
            while len(parts) > 1:
                parts = [parts[i] + parts[i + 1]
                         for i in range(0, len(parts) - 1, 2)] + (
                    [parts[-1]] if len(parts) % 2 else [])
            return parts[0]

        store_idx(0, 0)
        fire_a(0)
        fire_b(0)

        def chunk_body(c, carry):
            p = c & 1
            pltpu.sync_copy(feat_hbm.at[pl.ds(base + c * CHUNK, CHUNK)], fd_v)

            def s_body_a(s, carry2):
                fd = [fd_v[s, pl.ds(j * LANES, LANES)] for j in range(NFC)]
                for d in range(8):
                    r = d * CHUNK + s
                    acc = tree_sum([rows_a[r, pl.ds(LANES * j, LANES)] * fd[j]
                                    for j in range(NFC)])
                    out_v[s, pl.ds(d * LANES, LANES)] = acc
                return carry2

            rot = jnp.arange(1, LANES + 1, dtype=jnp.int32) % LANES

            def s_body_b(s, tvec):
                fd = [fd_v[s, pl.ds(j * LANES, LANES)] for j in range(NFC)]
                for d in range(8, NGATH):
                    r = (d - 8) * CHUNK + s
                    acc = tree_sum([rows_b[r, pl.ds(LANES * j, LANES)] * fd[j]
                                    for j in range(NFC)])
                    out_v[s, pl.ds(d * LANES, LANES)] = acc
                qs = tvec[0] + NUM_CLASSES
                for d in range(NGATH, DEPTH):
                    node = jnp.maximum((qs >> (d + 1)) - 1, 0)
                    acc = tree_sum([hot_v[node, pl.ds(LANES * j, LANES)] * fd[j]
                                    for j in range(NFC)])
                    out_v[s, pl.ds(d * LANES, LANES)] = acc
                return tvec.at[rot].get(mode="promise_in_bounds")

            pltpu.make_async_copy(table_hbm.at[idx_a.at[p]], rows_a, sem_a).wait()
            lax.fori_loop(0, CHUNK, s_body_a, 0)

            @pl.when(c + 1 < nchunk)
            def _():
                store_idx(c + 1, 1 - p)
                fire_a(1 - p)

            pltpu.make_async_copy(table_hbm.at[idx_b.at[p]], rows_b, sem_b).wait()
            lax.fori_loop(0, CHUNK, s_body_b,
                          tgt_v[pl.ds(c * CHUNK, CHUNK)])

            @pl.when(c + 1 < nchunk)
            def _():
                fire_b(1 - p)

            pltpu.sync_copy(out_v, out_hbm.at[pl.ds(base + c * CHUNK, CHUNK)])
            return carry

        lax.fori_loop(0, nchunk, chunk_body, 0)

    return sc_fn(features, tgt, wdiff)


def _selection_matrix():
    l = jnp.arange(DEPTH * LANES)
    return (l[:, None] // LANES == jnp.arange(DEPTH)[None, :]).astype(jnp.float32)


def _tc_finish(part, tgt2d, smat, batch):
    blk = 2048
    grid = (batch // blk,)

    def body(part_ref, tgt_ref, s_ref, out_ref):
        x = part_ref[...]
        z = jnp.dot(x, s_ref[...], preferred_element_type=jnp.float32)
        curr = tgt_ref[...] + NUM_INTERNAL
        sgs, ms = [], []
        for _ in range(DEPTH):
            valid = curr > 0
            cm1 = curr - 1
            sg = (1 - 2 * (cm1 & 1)).astype(jnp.float32)
            sgs.append(sg)
            ms.append(valid.astype(jnp.float32))
            curr = jnp.where(valid, cm1 >> 1, 0)
        sig = jnp.concatenate(sgs, axis=1)
        mk = jnp.concatenate(ms, axis=1)
        v = sig * z
        sp = jnp.maximum(v, 0.0) + jnp.log1p(jnp.exp(-jnp.abs(v)))
        out_ref[...] = jnp.sum(sp * mk, axis=1)

    return pl.pallas_call(
        body,
        grid=grid,
        in_specs=[
            pl.BlockSpec((blk, DEPTH * LANES), lambda i: (i, 0)),
            pl.BlockSpec((blk, 1), lambda i: (i, 0)),
            pl.BlockSpec((DEPTH * LANES, DEPTH), lambda i: (0, 0)),
        ],
        out_specs=pl.BlockSpec((blk,), lambda i: (i,)),
        out_shape=jax.ShapeDtypeStruct((batch,), jnp.float32),
    )(part, tgt2d, smat)


def kernel(features, targets, node_weights, path_nodes_map, path_directions_map):
    del path_nodes_map, path_directions_map
    batch, feat = features.shape
    tgt = targets.astype(jnp.int32)
    wdiff = node_weights[:, :, 1] - node_weights[:, :, 0]
    part = _sc_partials(features, tgt.reshape(batch), wdiff, batch)
    return _tc_finish(part, tgt, _selection_matrix(), batch)


# speedup vs baseline: 1.2468x; 1.2468x over previous
import functools

import jax
import jax.numpy as jnp
from jax import lax
from jax.experimental import pallas as pl
from jax.experimental.pallas import tpu as pltpu
from jax.experimental.pallas import tpu_sc as plsc

NUM_CLASSES = 100000
NUM_INTERNAL = NUM_CLASSES - 1
DEPTH = 17
FEAT = 128
LANES = 16
CHUNK = 16
NGATH = 9
NHOT = 256
NFC = FEAT // LANES


def _sc_partials(features, tgt, wdiff, batch):
    info = plsc.get_sparse_core_info()
    nc, ns = info.num_cores, info.num_subcores
    nw = nc * ns
    spw = batch // nw
    nchunk = spw // CHUNK
    na = 8 * CHUNK
    nb = (NGATH - 8) * CHUNK

    mesh = plsc.VectorSubcoreMesh(core_axis_name="c", subcore_axis_name="s")

    @functools.partial(
        pl.kernel,
        mesh=mesh,
        out_type=jax.ShapeDtypeStruct((batch, DEPTH * LANES), jnp.float32),
        scratch_types=[
            pltpu.VMEM((spw,), jnp.int32),
            pltpu.VMEM((2, na), jnp.int32),
            pltpu.VMEM((2, nb), jnp.int32),
            pltpu.VMEM((na, FEAT), jnp.float32),
            pltpu.VMEM((nb, FEAT), jnp.float32),
            pltpu.VMEM((NHOT, FEAT), jnp.float32),
            pltpu.VMEM((CHUNK, FEAT), jnp.float32),
            pltpu.VMEM((CHUNK, DEPTH * LANES), jnp.float32),
            pltpu.SemaphoreType.DMA,
            pltpu.SemaphoreType.DMA,
        ],
    )
    def sc_fn(feat_hbm, tgt_hbm, table_hbm, out_hbm,
              tgt_v, idx_a, idx_b, rows_a, rows_b, hot_v, fd_v, out_v,
              sem_a, sem_b):
        wid = lax.axis_index("s") * nc + lax.axis_index("c")
        base = wid * spw
        pltpu.sync_copy(tgt_hbm.at[pl.ds(base, spw)], tgt_v)
        pltpu.sync_copy(table_hbm.at[pl.ds(0, NHOT)], hot_v)

        def store_idx(c, p):
            q = tgt_v[pl.ds(c * CHUNK, CHUNK)] + NUM_CLASSES
            for d in range(NGATH):
                node = jnp.maximum((q >> (d + 1)) - 1, 0)
                if d < 8:
                    idx_a[p, pl.ds(d * CHUNK, CHUNK)] = node
                else:
                    idx_b[p, pl.ds((d - 8) * CHUNK, CHUNK)] = node

        def fire_a(p):
            return pltpu.async_copy(table_hbm.at[idx_a.at[p]], rows_a, sem_a)

        def fire_b(p):
            return pltpu.async_copy(table_hbm.at[idx_b.at[p]], rows_b, sem_b)

        def tree_sum(parts):
            while len(parts) > 1:
                parts = [parts[i] + parts[i + 1]
                         for i in range(0, len(parts) - 1, 2)] + (
                    [parts[-1]] if len(parts) % 2 else [])
            return parts[0]

        def node_terms(rows, r, fd):
            return tree_sum([rows[r, pl.ds(LANES * j, LANES)] * fd[j]
                             for j in range(NFC)])

        store_idx(0, 0)
        fire_a(0)
        fire_b(0)

        def chunk_body(c, carry):
            p = c & 1
            pltpu.sync_copy(feat_hbm.at[pl.ds(base + c * CHUNK, CHUNK)], fd_v)

            def s_body_a(s, carry2):
                fd = [fd_v[s, pl.ds(j * LANES, LANES)] for j in range(NFC)]
                for d in range(8):
                    out_v[s, pl.ds(d * LANES, LANES)] = node_terms(
                        rows_a, d * CHUNK + s, fd)
                return carry2

            rot = jnp.arange(1, LANES + 1, dtype=jnp.int32) % LANES

            def s_body_b(s, tvec):
                fd = [fd_v[s, pl.ds(j * LANES, LANES)] for j in range(NFC)]
                for d in range(8, NGATH):
                    out_v[s, pl.ds(d * LANES, LANES)] = node_terms(
                        rows_b, (d - 8) * CHUNK + s, fd)
                qs = tvec[0] + NUM_CLASSES
                for d in range(NGATH, DEPTH):
                    node = jnp.maximum((qs >> (d + 1)) - 1, 0)
                    out_v[s, pl.ds(d * LANES, LANES)] = node_terms(
                        hot_v, node, fd)
                return tvec.at[rot].get(mode="promise_in_bounds")

            pltpu.make_async_copy(table_hbm.at[idx_a.at[p]], rows_a, sem_a).wait()
            lax.fori_loop(0, CHUNK, s_body_a, 0)

            @pl.when(c + 1 < nchunk)
            def _():
                store_idx(c + 1, 1 - p)
                fire_a(1 - p)

            pltpu.make_async_copy(table_hbm.at[idx_b.at[p]], rows_b, sem_b).wait()
            lax.fori_loop(0, CHUNK, s_body_b,
                          tgt_v[pl.ds(c * CHUNK, CHUNK)])

            @pl.when(c + 1 < nchunk)
            def _():
                fire_b(1 - p)

            pltpu.sync_copy(out_v, out_hbm.at[pl.ds(base + c * CHUNK, CHUNK)])
            return carry

        lax.fori_loop(0, nchunk, chunk_body, 0)

    return sc_fn(features, tgt, wdiff)


def _selection_matrix():
    l = jnp.arange(DEPTH * LANES)
    return (l[:, None] // LANES == jnp.arange(DEPTH)[None, :]).astype(jnp.float32)


def _tc_finish(part, tgt2d, smat, batch):
    blk = 2048
    grid = (batch // blk,)

    def body(part_ref, tgt_ref, s_ref, out_ref):
        x = part_ref[...]
        z = jnp.dot(x, s_ref[...], preferred_element_type=jnp.float32)
        curr = tgt_ref[...] + NUM_INTERNAL
        sgs, ms = [], []
        for _ in range(DEPTH):
            valid = curr > 0
            cm1 = curr - 1
            sg = (1 - 2 * (cm1 & 1)).astype(jnp.float32)
            sgs.append(sg)
            ms.append(valid.astype(jnp.float32))
            curr = jnp.where(valid, cm1 >> 1, 0)
        sig = jnp.concatenate(sgs, axis=1)
        mk = jnp.concatenate(ms, axis=1)
        v = sig * z
        sp = jnp.maximum(v, 0.0) + jnp.log1p(jnp.exp(-jnp.abs(v)))
        out_ref[...] = jnp.sum(sp * mk, axis=1)

    return pl.pallas_call(
        body,
        grid=grid,
        in_specs=[
            pl.BlockSpec((blk, DEPTH * LANES), lambda i: (i, 0)),
            pl.BlockSpec((blk, 1), lambda i: (i, 0)),
            pl.BlockSpec((DEPTH * LANES, DEPTH), lambda i: (0, 0)),
        ],
        out_specs=pl.BlockSpec((blk,), lambda i: (i,)),
        out_shape=jax.ShapeDtypeStruct((batch,), jnp.float32),
    )(part, tgt2d, smat)


def kernel(features, targets, node_weights, path_nodes_map, path_directions_map):
    del path_nodes_map, path_directions_map
    batch, feat = features.shape
    tgt = targets.astype(jnp.int32)
    wdiff = jnp.tensordot(node_weights, jnp.asarray([-1.0, 1.0], jnp.float32),
                          axes=([2], [0]))
    part = _sc_partials(features, tgt.reshape(batch), wdiff, batch)
    return _tc_finish(part, tgt, _selection_matrix(), batch)
